# baseline (speedup 1.0000x reference)
import functools

import jax
import jax.numpy as jnp
from jax import lax
from jax.experimental import pallas as pl
from jax.experimental.pallas import tpu as pltpu
from jax.experimental.pallas import tpu_sc as plsc

TOTAL_ENT = 1000000
TOTAL_REL = 1000
EMB_DIM = 64
MARGIN = 1.0
BATCH = 16384

NC = 2
NS = 16
NW = NC * NS
BPW = BATCH // NW
CH = 128
NCH = BPW // CH


def _score_kernel_body(ent_hbm, rel_hbm, hidx_hbm, tidx_hbm, ridx_hbm,
                       score_hbm,
                       hidx_v, tidx_v, ridx_v, h_rows, t_rows, r_rows,
                       score_v, sem):
    wid = lax.axis_index("s") * NC + lax.axis_index("c")

    pltpu.sync_copy(hidx_hbm.at[wid], hidx_v)
    pltpu.sync_copy(tidx_hbm.at[wid], tidx_v)
    pltpu.sync_copy(ridx_hbm.at[wid], ridx_v)

    copies = []
    for j in range(NCH):
        sl = pl.ds(j * CH, CH)
        copies.append(pltpu.async_copy(ent_hbm.at[hidx_v.at[j]], h_rows.at[sl], sem))
        copies.append(pltpu.async_copy(ent_hbm.at[tidx_v.at[j]], t_rows.at[sl], sem))
        copies.append(pltpu.async_copy(rel_hbm.at[ridx_v.at[j]], r_rows.at[sl], sem))
    for c in copies:
        c.wait()

    lane = lax.iota(jnp.int32, 16)

    def blk_body(blk, carry):
        rows = blk * 16 + lane

        def d_body(d, acc):
            cols = jnp.zeros((16,), jnp.int32) + d
            hv = plsc.load_gather(h_rows, [rows, cols])
            tv = plsc.load_gather(t_rows, [rows, cols])
            rv = plsc.load_gather(r_rows, [rows, cols])
            return acc + hv * tv * rv

        acc = lax.fori_loop(0, EMB_DIM, d_body, jnp.zeros((16,), jnp.float32))
        score_v[pl.ds(blk * 16, 16)] = acc
        return carry

    lax.fori_loop(0, BPW // 16, blk_body, 0)

    pltpu.sync_copy(score_v, score_hbm.at[pl.ds(wid * BPW, BPW)])


_score_kernel = functools.partial(
    pl.kernel,
    out_type=jax.ShapeDtypeStruct((BATCH,), jnp.float32),
    mesh=plsc.VectorSubcoreMesh(core_axis_name="c", subcore_axis_name="s"),
    compiler_params=pltpu.CompilerParams(
        needs_layout_passes=False, use_tc_tiling_on_sc=False),
    scratch_types=[
        pltpu.VMEM((NCH, CH), jnp.int32),
        pltpu.VMEM((NCH, CH), jnp.int32),
        pltpu.VMEM((NCH, CH), jnp.int32),
        pltpu.VMEM((BPW, EMB_DIM), jnp.float32),
        pltpu.VMEM((BPW, EMB_DIM), jnp.float32),
        pltpu.VMEM((BPW, EMB_DIM), jnp.float32),
        pltpu.VMEM((BPW,), jnp.float32),
        pltpu.SemaphoreType.DMA,
    ],
)(_score_kernel_body)


def _loss_body(pos_ref, neg_ref, out_ref):
    out_ref[0, 0] = jnp.sum(
        jnp.maximum(pos_ref[:, :] - neg_ref[:, :] + MARGIN, 0.0))


_loss_call = pl.pallas_call(
    _loss_body,
    out_shape=jax.ShapeDtypeStruct((1, 1), jnp.float32),
    out_specs=pl.BlockSpec(memory_space=pltpu.SMEM),
)


def kernel(batch_h, batch_t, batch_r, batch_y, ent_embeddings, rel_embeddings):
    hidx = batch_h.reshape(NW, NCH, CH)
    tidx = batch_t.reshape(NW, NCH, CH)
    ridx = batch_r.reshape(NW, NCH, CH)
    score = _score_kernel(ent_embeddings, rel_embeddings, hidx, tidx, ridx)
    half = BATCH // 2
    pos_score = score[:half]
    neg_score = score[half:]
    loss = _loss_call(pos_score.reshape(64, 128), neg_score.reshape(64, 128))[0, 0]
    return (loss, pos_score, neg_score)
